# baseline (speedup 1.0000x reference)
import functools

import jax
import jax.numpy as jnp
from jax import lax
from jax.experimental import pallas as pl
from jax.experimental.pallas import tpu as pltpu
from jax.experimental.pallas import tpu_sc as plsc

B = 16384
D = 64
NC = 2
NS = 16
NW = NC * NS
BPW = B // NW
CH = 128
NCH = BPW // CH


def _body(wridx_hbm, cridx_hbm, wr_hbm, cr_hbm, out_hbm,
          idx_v, cidx_v, pair_v, sem, csem):
    wid = lax.axis_index("s") * NC + lax.axis_index("c")
    base = wid * BPW
    pltpu.sync_copy(wridx_hbm.at[wid], idx_v)
    pltpu.sync_copy(cridx_hbm.at[wid], cidx_v)
    for j in range(NCH):
        pltpu.async_copy(wr_hbm.at[idx_v.at[j]],
                         pair_v.at[pl.ds(j * CH, CH)], sem)
    pltpu.make_async_copy(wr_hbm.at[pl.ds(0, BPW)], pair_v, sem).wait()
    pltpu.sync_copy(pair_v, out_hbm.at[0, pl.ds(base, BPW)])
    for j in range(NCH):
        pltpu.async_copy(cr_hbm.at[cidx_v.at[j]],
                         pair_v.at[pl.ds(j * CH, CH)], csem)
    pltpu.make_async_copy(cr_hbm.at[pl.ds(0, BPW)], pair_v, csem).wait()
    pltpu.sync_copy(pair_v, out_hbm.at[1, pl.ds(base, BPW)])


@jax.jit
def _lookup(wridx, cridx, wr, cr):
    mesh = plsc.VectorSubcoreMesh(core_axis_name="c", subcore_axis_name="s")
    run = functools.partial(
        pl.kernel,
        mesh=mesh,
        out_type=jax.ShapeDtypeStruct((2, B, 2 * D), jnp.float32),
        scratch_types=[
            pltpu.VMEM((NCH, CH), jnp.int32),
            pltpu.VMEM((NCH, CH), jnp.int32),
            pltpu.VMEM((BPW, 2 * D), jnp.float32),
            pltpu.SemaphoreType.DMA,
            pltpu.SemaphoreType.DMA,
        ],
    )(_body)
    return run(wridx, cridx, wr, cr)


def kernel(words, contexts, w_table, c_table):
    wridx = (words.astype(jnp.int32) >> 1).reshape(NW, NCH, CH)
    cridx = (contexts.astype(jnp.int32) >> 1).reshape(NW, NCH, CH)
    return _lookup(wridx, cridx,
                   w_table.reshape(-1, 2 * D), c_table.reshape(-1, 2 * D))


# speedup vs baseline: 1.3727x; 1.3727x over previous
import functools

import jax
import jax.numpy as jnp
from jax import lax
from jax.experimental import pallas as pl
from jax.experimental.pallas import tpu as pltpu
from jax.experimental.pallas import tpu_sc as plsc

B = 16384
D = 64
NC = 2
NS = 16
NW = NC * NS
BPW = B // NW
CH = 32
NCH = BPW // CH
L = 16
G = 8

_LANE = None


def _gather_table(tab_hbm, idx_hbm, wid, out2d_hbm, base,
                  idx_v, tiles_v, outt_v, sem):
    pltpu.sync_copy(idx_hbm.at[wid], idx_v)
    lane = lax.iota(jnp.int32, L)

    def per_chunk(k, _):
        vecs = [idx_v[pl.ds(k * CH + g * L, L)] for g in range(CH // L)]
        copies = []
        for m in range(CH):
            v = vecs[m // L]
            i = jnp.sum(jnp.where(lane == (m % L), v, 0))
            row = pl.multiple_of((i >> 3) * G, G)
            copies.append(pltpu.async_copy(
                tab_hbm.at[pl.ds(row, G)],
                tiles_v.at[pl.ds(m * G, G)], sem))
        for cp in copies:
            cp.wait()
        for g in range(CH // L):
            jvec = (g * L + lane) * G + (vecs[g] & 7)
            for d in range(D):
                vals = plsc.load_gather(tiles_v, [jvec, lane * 0 + d])
                outt_v[d, pl.ds(k * CH + g * L, L)] = vals
        return _
    lax.fori_loop(0, NCH, per_chunk, 0)

    pltpu.sync_copy(outt_v, out2d_hbm.at[:, pl.ds(base, BPW)])


def _body(widx_hbm, cidx_hbm, w_hbm, c_hbm, out_hbm,
          idx_v, tiles_v, outt_v, sem):
    wid = lax.axis_index("s") * NC + lax.axis_index("c")
    base = wid * BPW
    _gather_table(w_hbm, widx_hbm, wid, out_hbm.at[0], base,
                  idx_v, tiles_v, outt_v, sem)
    _gather_table(c_hbm, cidx_hbm, wid, out_hbm.at[1], base,
                  idx_v, tiles_v, outt_v, sem)


@jax.jit
def _lookup(widx, cidx, w_table, c_table):
    mesh = plsc.VectorSubcoreMesh(core_axis_name="c", subcore_axis_name="s")
    run = functools.partial(
        pl.kernel,
        mesh=mesh,
        out_type=jax.ShapeDtypeStruct((2, D, B), jnp.float32),
        scratch_types=[
            pltpu.VMEM((BPW,), jnp.int32),
            pltpu.VMEM((CH * G, D), jnp.float32),
            pltpu.VMEM((D, BPW), jnp.float32),
            pltpu.SemaphoreType.DMA,
        ],
        compiler_params=pltpu.CompilerParams(needs_layout_passes=False),
    )(_body)
    out_t = run(widx, cidx, w_table, c_table)
    return out_t.transpose(0, 2, 1)


def kernel(words, contexts, w_table, c_table):
    widx = words.astype(jnp.int32).reshape(NW, BPW)
    cidx = contexts.astype(jnp.int32).reshape(NW, BPW)
    return _lookup(widx, cidx, w_table, c_table)


# speedup vs baseline: 1.4497x; 1.0561x over previous
import functools

import jax
import jax.numpy as jnp
from jax import lax
from jax.experimental import pallas as pl
from jax.experimental.pallas import tpu as pltpu
from jax.experimental.pallas import tpu_sc as plsc

B = 16384
D = 64
NC = 2
NS = 16
NW = NC * NS
BPW = B // NW
CH = 32
NCH = BPW // CH
L = 16
G = 8

_LANE = None


def _gather_table(tab_hbm, idx_hbm, wid, out2d_hbm, base,
                  idx_v, tiles_v, outt_v, sem):
    pltpu.sync_copy(idx_hbm.at[wid], idx_v)
    lane = lax.iota(jnp.int32, L)

    def per_chunk(k, _):
        vecs = [idx_v[pl.ds(k * CH + g * L, L)] for g in range(CH // L)]
        copies = []
        for m in range(CH):
            v = vecs[m // L]
            i = jnp.sum(jnp.where(lane == (m % L), v, 0))
            row = pl.multiple_of((i >> 3) * G, G)
            copies.append(pltpu.async_copy(
                tab_hbm.at[pl.ds(row, G)],
                tiles_v.at[pl.ds(m * G, G)], sem))
        for cp in copies:
            cp.wait()
        for g in range(CH // L):
            jvec = (g * L + lane) * G + (vecs[g] & 7)
            for d in range(D):
                vals = plsc.load_gather(tiles_v, [jvec, lane * 0 + d])
                outt_v[d, pl.ds(k * CH + g * L, L)] = vals
        return _
    lax.fori_loop(0, NCH, per_chunk, 0)

    pltpu.sync_copy(outt_v, out2d_hbm.at[:, pl.ds(base, BPW)])


def _body(idx_hbm, tab_hbm, out_hbm, idx_v, tiles_v, outt_v, sem):
    wid = lax.axis_index("s") * NC + lax.axis_index("c")
    base = wid * BPW
    _gather_table(tab_hbm, idx_hbm, wid, out_hbm, base,
                  idx_v, tiles_v, outt_v, sem)


@jax.jit
def _lookup(widx, cidx, w_table, c_table):
    mesh = plsc.VectorSubcoreMesh(core_axis_name="c", subcore_axis_name="s")
    run = functools.partial(
        pl.kernel,
        mesh=mesh,
        out_type=jax.ShapeDtypeStruct((D, B), jnp.float32),
        scratch_types=[
            pltpu.VMEM((BPW,), jnp.int32),
            pltpu.VMEM((CH * G, D), jnp.float32),
            pltpu.VMEM((D, BPW), jnp.float32),
            pltpu.SemaphoreType.DMA,
        ],
        compiler_params=pltpu.CompilerParams(needs_layout_passes=False),
    )(_body)
    out_w = run(widx, w_table)
    out_c = run(cidx, c_table)
    out_t = jnp.stack((out_w, out_c), axis=0)
    return out_t.transpose(0, 2, 1)


def kernel(words, contexts, w_table, c_table):
    widx = words.astype(jnp.int32).reshape(NW, BPW)
    cidx = contexts.astype(jnp.int32).reshape(NW, BPW)
    return _lookup(widx, cidx, w_table, c_table)


# speedup vs baseline: 1.4517x; 1.0013x over previous
import functools

import jax
import jax.numpy as jnp
from jax import lax
from jax.experimental import pallas as pl
from jax.experimental.pallas import tpu as pltpu
from jax.experimental.pallas import tpu_sc as plsc

B = 16384
D = 64
NC = 2
NS = 16
NW = NC * NS
BPW = B // NW
CH = 32
NCH = BPW // CH
L = 16
G = 8


def _gather_table(tab_hbm, idx_hbm, wid, out2d_hbm, base,
                  idx_v, tiles_v, outt_v, sem):
    pltpu.sync_copy(idx_hbm.at[wid], idx_v)
    lane = lax.iota(jnp.int32, L)

    def per_chunk(k, _):
        vecs = [idx_v[pl.ds(k * CH + g * L, L)] for g in range(CH // L)]
        copies = []
        for m in range(CH):
            v = vecs[m // L]
            i = jnp.sum(jnp.where(lane == (m % L), v, 0))
            row = pl.multiple_of((i >> 3) * G, G)
            copies.append(pltpu.async_copy(
                tab_hbm.at[pl.ds(row, G)],
                tiles_v.at[pl.ds(m * G, G)], sem))
        for cp in copies:
            cp.wait()
        for g in range(CH // L):
            jvec = (g * L + lane) * G + (vecs[g] & 7)
            for d in range(D):
                vals = plsc.load_gather(tiles_v, [jvec, lane * 0 + d])
                outt_v[d, pl.ds(k * CH + g * L, L)] = vals
        return _
    lax.fori_loop(0, NCH, per_chunk, 0)

    pltpu.sync_copy(outt_v, out2d_hbm.at[:, pl.ds(base, BPW)])


def _body(idx_hbm, tab_hbm, out_hbm, idx_v, tiles_v, outt_v, sem):
    wid = lax.axis_index("s") * NC + lax.axis_index("c")
    base = wid * BPW
    _gather_table(tab_hbm, idx_hbm, wid, out_hbm, base,
                  idx_v, tiles_v, outt_v, sem)


@jax.jit
def _lookup(widx, cidx, w_table, c_table):
    mesh = plsc.VectorSubcoreMesh(core_axis_name="c", subcore_axis_name="s")
    run = functools.partial(
        pl.kernel,
        mesh=mesh,
        out_type=jax.ShapeDtypeStruct((D, B), jnp.float32),
        scratch_types=[
            pltpu.VMEM((BPW,), jnp.int32),
            pltpu.VMEM((CH * G, D), jnp.float32),
            pltpu.VMEM((D, BPW), jnp.float32),
            pltpu.SemaphoreType.DMA,
        ],
        compiler_params=pltpu.CompilerParams(needs_layout_passes=False),
    )(_body)
    out_w = run(widx, w_table)
    out_c = run(cidx, c_table)
    out_t = jnp.stack((out_w, out_c), axis=0)
    return out_t.transpose(0, 2, 1)


def kernel(words, contexts, w_table, c_table):
    widx = words.astype(jnp.int32).reshape(NW, BPW)
    cidx = contexts.astype(jnp.int32).reshape(NW, BPW)
    return _lookup(widx, cidx, w_table, c_table)
